# baseline (speedup 1.0000x reference)
import functools

import jax
import jax.numpy as jnp
from jax.experimental import pallas as pl

_K = 1024
_D = 64
_COMMITMENT_COST = 0.25


def _vq_block_kernel(x_ref, emb_ref, q_ref, idx_ref, loss_ref, ppl_ref,
                     *, n_rows: int):
    i = pl.program_id(0)
    nb = pl.num_programs(0)

    xb = x_ref[...]
    emb = emb_ref[...]

    @pl.when(i == 0)
    def _init():
        loss_ref[...] = jnp.zeros_like(loss_ref)
        ppl_ref[...] = jnp.zeros_like(ppl_ref)

    a = jnp.sum(xb * xb, axis=1, keepdims=True)
    at = a.T
    b = jnp.sum(emb * emb, axis=1, keepdims=True)
    mmt = jax.lax.dot_general(
        emb, xb, (((1,), (1,)), ((), ())),
        preferred_element_type=jnp.float32)
    distt = (at + b) - 2.0 * mmt

    m = jnp.min(distt, axis=0, keepdims=True)
    rowids = jax.lax.broadcasted_iota(jnp.int32, distt.shape, 0)
    idx = jnp.min(jnp.where(distt == m, rowids, _K), axis=0)
    idx_ref[...] = idx[:, None]

    onehott = (rowids == idx[None, :]).astype(jnp.float32)
    q = jax.lax.dot_general(
        onehott, emb, (((0,), (0,)), ((), ())),
        preferred_element_type=jnp.float32)
    q_ref[...] = q

    loss_ref[...] += jnp.sum(m, axis=1, keepdims=True)

    @pl.when(i == nb - 1)
    def _finalize():
        mse = loss_ref[...] / (n_rows * _D)
        loss_ref[...] = mse + _COMMITMENT_COST * mse
        avg = loss_ref[...] * 0.0 + (1.0 / _K)
        ppl_ref[...] = jnp.exp(-(avg * jnp.log(avg + 1e-10)))


def kernel(x, emb_weight):
    n_rows = x.shape[0] * x.shape[1]
    flat = x.reshape(n_rows, _D)
    blk = 2304
    nb = n_rows // blk

    q, idx, loss, ppl = pl.pallas_call(
        functools.partial(_vq_block_kernel, n_rows=n_rows),
        grid=(nb,),
        in_specs=[
            pl.BlockSpec((blk, _D), lambda i: (i, 0)),
            pl.BlockSpec((_K, _D), lambda i: (0, 0)),
        ],
        out_specs=[
            pl.BlockSpec((blk, _D), lambda i: (i, 0)),
            pl.BlockSpec((blk, 1), lambda i: (i, 0)),
            pl.BlockSpec((1, 1), lambda i: (0, 0)),
            pl.BlockSpec((1, 1), lambda i: (0, 0)),
        ],
        out_shape=[
            jax.ShapeDtypeStruct((n_rows, _D), jnp.float32),
            jax.ShapeDtypeStruct((n_rows, 1), jnp.int32),
            jax.ShapeDtypeStruct((1, 1), jnp.float32),
            jax.ShapeDtypeStruct((1, 1), jnp.float32),
        ],
    )(flat, emb_weight)

    return (q.reshape(x.shape), loss[0, 0], ppl[0, 0], idx)


# speedup vs baseline: 1.0713x; 1.0713x over previous
import functools

import jax
import jax.numpy as jnp
from jax.experimental import pallas as pl
from jax.experimental.pallas import tpu as pltpu

_K = 1024
_D = 64
_COMMITMENT_COST = 0.25


def _vq_block_kernel(x_ref, emb_ref, embt_ref,
                     q_ref, idx_ref, cnt_ref, loss_ref, ppl_ref,
                     b_ref,
                     *, n_rows: int):
    i = pl.program_id(0)
    nb = pl.num_programs(0)

    xb = x_ref[...]
    emb = emb_ref[...]
    embt = embt_ref[...]

    @pl.when(i == 0)
    def _precompute():
        b_ref[...] = jnp.sum(embt * embt, axis=0, keepdims=True)
        loss_ref[...] = jnp.zeros_like(loss_ref)
        cnt_ref[...] = jnp.zeros_like(cnt_ref)
        ppl_ref[...] = jnp.zeros_like(ppl_ref)

    a = jnp.sum(xb * xb, axis=1, keepdims=True)
    b = b_ref[...]
    mm = jax.lax.dot_general(
        xb, embt, (((1,), (0,)), ((), ())),
        preferred_element_type=jnp.float32)
    dist = (a + b) - 2.0 * mm

    m = jnp.min(dist, axis=1, keepdims=True)
    colids = jax.lax.broadcasted_iota(jnp.int32, dist.shape, 1)
    idx = jnp.min(jnp.where(dist == m, colids, _K), axis=1)
    idx_ref[...] = idx.reshape(idx_ref.shape)

    onehot = (colids == idx[:, None]).astype(jnp.float32)
    q = jax.lax.dot_general(
        onehot, emb, (((1,), (0,)), ((), ())),
        preferred_element_type=jnp.float32)
    q_ref[...] = q

    loss_ref[...] += jnp.sum(m, axis=0, keepdims=True).reshape(1, 1)

    @pl.when(i == nb - 1)
    def _finalize():
        mse = loss_ref[...] / (n_rows * _D)
        loss_ref[...] = mse + _COMMITMENT_COST * mse
        avg = cnt_ref[...][:, :1] * 0.0 + (1.0 / _K)
        ppl_ref[...] = jnp.exp(-(avg * jnp.log(avg + 1e-10)))


def kernel(x, emb_weight):
    n_rows = x.shape[0] * x.shape[1]
    flat = x.reshape(n_rows, _D)
    blk = 2304
    nb = n_rows // blk

    q, idx, _cnt, loss, ppl = pl.pallas_call(
        functools.partial(_vq_block_kernel, n_rows=n_rows),
        grid=(nb,),
        in_specs=[
            pl.BlockSpec((blk, _D), lambda i: (i, 0)),
            pl.BlockSpec((_K, _D), lambda i: (0, 0)),
            pl.BlockSpec((_D, _K), lambda i: (0, 0)),
        ],
        out_specs=[
            pl.BlockSpec((blk, _D), lambda i: (i, 0)),
            pl.BlockSpec((1, blk // 128, 128), lambda i: (i, 0, 0)),
            pl.BlockSpec((1, _K), lambda i: (0, 0)),
            pl.BlockSpec((1, 1), lambda i: (0, 0)),
            pl.BlockSpec((1, 1), lambda i: (0, 0)),
        ],
        out_shape=[
            jax.ShapeDtypeStruct((n_rows, _D), jnp.float32),
            jax.ShapeDtypeStruct((nb, blk // 128, 128), jnp.int32),
            jax.ShapeDtypeStruct((1, _K), jnp.float32),
            jax.ShapeDtypeStruct((1, 1), jnp.float32),
            jax.ShapeDtypeStruct((1, 1), jnp.float32),
        ],
        scratch_shapes=[pltpu.VMEM((1, _K), jnp.float32)],
    )(flat, emb_weight, emb_weight.T)

    return (q.reshape(x.shape), loss[0, 0], ppl[0, 0],
            idx.reshape(n_rows, 1))
